# baseline (speedup 1.0000x reference)
import functools

import jax
import jax.numpy as jnp
from jax import lax
from jax.experimental import pallas as pl
from jax.experimental.pallas import tpu as pltpu
from jax.experimental.pallas import tpu_sc as plsc

N = 10000
E = 320000
G = 16
DIM_IN = 128
DN = 16

NW = 32
EPW = E // NW
CH = 2000
NCH = EPW // CH

NB_N = 10
BN = N // NB_N
BE = 3200
NBE = E // BE

NP = 10112


def _f32(x):
    return jnp.dot(x[0], x[1], preferred_element_type=jnp.float32)


def _h0_body(x_ref, w_ref, b_ref, o_ref):
    o_ref[...] = jnp.dot(x_ref[...], w_ref[...],
                         preferred_element_type=jnp.float32) + b_ref[...]


def _h0(xp, W1B, b1B):
    full = lambda a: pl.BlockSpec(a.shape, lambda: tuple(0 for _ in a.shape))
    return pl.pallas_call(
        _h0_body,
        in_specs=[full(xp), full(W1B), full(b1B)],
        out_specs=pl.BlockSpec((N // 8, 128), lambda: (0, 0)),
        out_shape=jax.ShapeDtypeStruct((N // 8, 128), jnp.float32),
    )(xp, W1B, b1B)


def _split(a):
    hi = a.astype(jnp.bfloat16)
    lo = (a - hi.astype(jnp.float32)).astype(jnp.bfloat16)
    return hi, lo


def _dot3(ahi, alo, bhi, blo):
    f = jnp.float32
    return (jnp.dot(ahi, bhi, preferred_element_type=f)
            + jnp.dot(ahi, blo, preferred_element_type=f)
            + jnp.dot(alo, bhi, preferred_element_type=f))


def _l3_w(a2, wk3hi, wk3lo, bk3):
    ahi, alo = _split(a2)
    parts = [
        _dot3(ahi[:, e * 96:(e + 1) * 96], alo[:, e * 96:(e + 1) * 96],
              wk3hi[...], wk3lo[...])
        for e in range(8)
    ]
    return (jnp.concatenate(parts, axis=1) + bk3[...]).astype(jnp.bfloat16)


def _mlp_msg_body(ea_ref, hs_ref, wk1hi, wk1lo, bk1, wk2hi, wk2lo, bk2,
                  wk3hi, wk3lo, bk3, S, R, a2_out, msg_out):
    bf = jnp.bfloat16
    eahi, ealo = _split(ea_ref[...])
    a1 = jnp.maximum(
        _dot3(eahi, ealo, wk1hi[...], wk1lo[...]) + bk1[...], 0.0)
    a1hi, a1lo = _split(a1)
    a2 = jnp.maximum(
        _dot3(a1hi, a1lo, wk2hi[...], wk2lo[...]) + bk2[...], 0.0)
    a2_out[...] = a2
    w = _l3_w(a2, wk3hi, wk3lo, bk3)
    hsbig = jnp.dot(hs_ref[...].astype(bf), S[...],
                    preferred_element_type=jnp.float32)
    msg_out[...] = jnp.dot((hsbig * w).astype(bf), R[...],
                           preferred_element_type=jnp.float32)


def _mlp_msg(ea, hs, Wk1hi, Wk1lo, bk1r, Wk2hi, Wk2lo, bk2r, Wk3hi, Wk3lo,
             bk3r, S, R):
    full = lambda a: pl.BlockSpec(a.shape, lambda i: tuple(0 for _ in a.shape))
    return pl.pallas_call(
        _mlp_msg_body,
        grid=(NBE,),
        in_specs=[
            pl.BlockSpec((BE // 8, 128), lambda i: (i, 0)),
            pl.BlockSpec((BE // 8, 128), lambda i: (i, 0)),
            full(Wk1hi), full(Wk1lo), full(bk1r), full(Wk2hi), full(Wk2lo),
            full(bk2r), full(Wk3hi), full(Wk3lo), full(bk3r), full(S),
            full(R),
        ],
        out_specs=[
            pl.BlockSpec((BE // 8, 8 * 96), lambda i: (i, 0)),
            pl.BlockSpec((BE // 8, 128), lambda i: (i, 0)),
        ],
        out_shape=[
            jax.ShapeDtypeStruct((E // 8, 8 * 96), jnp.float32),
            jax.ShapeDtypeStruct((E // 8, 128), jnp.float32),
        ],
    )(ea, hs, Wk1hi, Wk1lo, bk1r, Wk2hi, Wk2lo, bk2r, Wk3hi, Wk3lo, bk3r,
      S, R)


def _msg_body(a2_ref, hs_ref, wk3hi, wk3lo, bk3, S, R, msg_out):
    bf = jnp.bfloat16
    w = _l3_w(a2_ref[...], wk3hi, wk3lo, bk3)
    hsbig = jnp.dot(hs_ref[...].astype(bf), S[...],
                    preferred_element_type=jnp.float32)
    msg_out[...] = jnp.dot((hsbig * w).astype(bf), R[...],
                           preferred_element_type=jnp.float32)


def _msg(a2, hs, Wk3hi, Wk3lo, bk3r, S, R):
    full = lambda a: pl.BlockSpec(a.shape, lambda i: tuple(0 for _ in a.shape))
    return pl.pallas_call(
        _msg_body,
        grid=(NBE,),
        in_specs=[
            pl.BlockSpec((BE // 8, 8 * 96), lambda i: (i, 0)),
            pl.BlockSpec((BE // 8, 128), lambda i: (i, 0)),
            full(Wk3hi), full(Wk3lo), full(bk3r), full(S), full(R),
        ],
        out_specs=pl.BlockSpec((BE // 8, 128), lambda i: (i, 0)),
        out_shape=jax.ShapeDtypeStruct((E // 8, 128), jnp.float32),
    )(a2, hs, Wk3hi, Wk3lo, bk3r, S, R)


def _update_body(s_ref, c_ref, h_ref, root, cb, o_ref):
    cnt = jnp.maximum(c_ref[0] + c_ref[1], 1.0)
    aggr = (s_ref[0] + s_ref[1]) / cnt
    hr = jnp.dot(h_ref[...], root[...], preferred_element_type=jnp.float32)
    o_ref[...] = jnp.maximum(aggr + hr + cb[...], 0.0)


def _update(sp, cp, hp, rootB, cbB):
    full = lambda a: pl.BlockSpec(a.shape, lambda: tuple(0 for _ in a.shape))
    return pl.pallas_call(
        _update_body,
        in_specs=[full(sp), full(cp), full(hp), full(rootB), full(cbB)],
        out_specs=pl.BlockSpec((N // 8, 128), lambda: (0, 0)),
        out_shape=jax.ShapeDtypeStruct((N // 8, 128), jnp.float32),
    )(sp, cp, hp, rootB, cbB)


def _pool_body(h_ref, b_ref, w2, b2, o_ref):
    ids = lax.broadcasted_iota(jnp.int32, (G, NP), 0)
    oh = (ids == b_ref[...]).astype(jnp.float32)
    s = jnp.dot(oh, h_ref[...], preferred_element_type=jnp.float32)
    cnt = jnp.maximum(jnp.sum(oh, axis=1, keepdims=True), 1.0)
    o_ref[...] = jnp.dot(s / cnt, w2[...],
                         preferred_element_type=jnp.float32) + b2[...]


def _pool(hp, bp, W2, b2r):
    full = lambda a: pl.BlockSpec(a.shape, lambda: tuple(0 for _ in a.shape))
    return pl.pallas_call(
        _pool_body,
        in_specs=[full(hp), full(bp), full(W2), full(b2r)],
        out_specs=pl.BlockSpec((G, 1), lambda: (0, 0)),
        out_shape=jax.ShapeDtypeStruct((G, 1), jnp.float32),
    )(hp, bp, W2, b2r)


_MESH = plsc.VectorSubcoreMesh(core_axis_name="c", subcore_axis_name="s")
_SC_PARAMS = pltpu.CompilerParams(use_tc_tiling_on_sc=False)


@functools.partial(
    pl.kernel,
    out_type=jax.ShapeDtypeStruct((E, DN), jnp.float32),
    mesh=_MESH,
    compiler_params=_SC_PARAMS,
    scratch_types=[
        pltpu.VMEM((CH,), jnp.int32),
        pltpu.VMEM((CH, DN), jnp.float32),
        pltpu.SemaphoreType.DMA,
    ],
)
def _gather_k(h_hbm, src_hbm, out_hbm, idx_v, rows_v, sem):
    cid = lax.axis_index("c")
    sid = lax.axis_index("s")
    wid = sid * 2 + cid
    for j in range(NCH):
        base = wid * EPW + j * CH
        pltpu.sync_copy(src_hbm.at[pl.ds(base, CH)], idx_v)
        pltpu.async_copy(h_hbm.at[idx_v], rows_v, sem).wait()
        pltpu.sync_copy(rows_v, out_hbm.at[pl.ds(base, CH)])


def _make_scatter(with_cnt):
    outs = (jax.ShapeDtypeStruct((2, N, DN), jnp.float32),)
    scratch = [
        pltpu.VMEM((CH,), jnp.int32),
        pltpu.VMEM((CH, DN), jnp.float32),
        pltpu.VMEM_SHARED((N, DN), jnp.float32),
    ]
    if with_cnt:
        outs = outs + (jax.ShapeDtypeStruct((2, N, DN), jnp.float32),)
        scratch += [
            pltpu.VMEM((CH, DN), jnp.float32),
            pltpu.VMEM_SHARED((N, DN), jnp.float32),
        ]

    @functools.partial(pl.kernel, out_type=outs, mesh=_MESH,
                       compiler_params=_SC_PARAMS, scratch_types=scratch)
    def _scatter_k(msg_hbm, dst_hbm, zeros_hbm, ones_hbm, *rest):
        if with_cnt:
            s_out, c_out, idx_v, rows_v, s_sh, ones_v, c_sh = rest
        else:
            s_out, idx_v, rows_v, s_sh = rest
        cid = lax.axis_index("c")
        sid = lax.axis_index("s")
        wid = sid * 2 + cid

        @pl.when(sid == 0)
        def _():
            pltpu.sync_copy(zeros_hbm, s_sh)
            if with_cnt:
                pltpu.sync_copy(zeros_hbm, c_sh)

        if with_cnt:
            pltpu.sync_copy(ones_hbm, ones_v)
        plsc.subcore_barrier()
        for j in range(NCH):
            base = wid * EPW + j * CH
            pltpu.sync_copy(dst_hbm.at[pl.ds(base, CH)], idx_v)
            pltpu.sync_copy(msg_hbm.at[pl.ds(base, CH)], rows_v)
            pltpu.sync_copy(rows_v, s_sh.at[idx_v], add=True)
            if with_cnt:
                pltpu.sync_copy(ones_v, c_sh.at[idx_v], add=True)
        plsc.subcore_barrier()

        @pl.when(sid == 0)
        def _():
            pltpu.sync_copy(s_sh, s_out.at[cid])
            if with_cnt:
                pltpu.sync_copy(c_sh, c_out.at[cid])

    return _scatter_k


_scatter_cnt_k = _make_scatter(True)
_scatter_k = _make_scatter(False)


def kernel(x, edge_index, edge_attr, batch, W1, b1, Wk1, bk1, Wk2, bk2,
           Wk3, bk3, root, cbias, W2, b2):
    src = edge_index[0]
    dst = edge_index[1]

    b2r = b2.reshape(1, 1)

    j = jnp.arange(DN * DN)
    S0 = (j[None, :] // DN == jnp.arange(DN)[:, None]).astype(jnp.float32)
    R0 = (j[:, None] % DN == jnp.arange(DN)[None, :]).astype(jnp.float32)
    I8 = jnp.eye(8, dtype=jnp.float32)
    kr = lambda W: jnp.kron(I8, W).astype(jnp.bfloat16)
    S = kr(S0)
    R = kr(R0)
    def split(a):
        hi = a.astype(jnp.bfloat16)
        lo = (a - hi.astype(jnp.float32)).astype(jnp.bfloat16)
        return hi, lo

    Wk1hi, Wk1lo = split(jnp.kron(I8, Wk1))
    Wk2hi, Wk2lo = split(jnp.kron(I8, Wk2))
    Wk3hi, Wk3lo = split(Wk3)
    bk1r = jnp.tile(bk1, 8).reshape(1, 8 * 64)
    bk2r = jnp.tile(bk2, 8).reshape(1, 8 * 96)
    bk3r = jnp.tile(bk3, 8).reshape(1, 8 * DN * DN)
    W1B = jnp.kron(I8, W1)
    b1B = jnp.tile(b1, 8).reshape(1, 128)
    rootB = jnp.kron(I8, root)
    cbB = jnp.tile(cbias, 8).reshape(1, 128)

    zeros = jnp.zeros((N // 8, 128), jnp.float32).reshape(N, DN)
    ones = jnp.ones((CH // 8, 128), jnp.float32).reshape(CH, DN)

    eaP = edge_attr.reshape(E // 8, 128)
    xp = x.reshape(N // 8, 8 * DIM_IN)
    h0 = _h0(xp, W1B, b1B)
    hs0 = _gather_k(h0.reshape(N, DN), src).reshape(E // 8, 128)
    a2, msg1 = _mlp_msg(eaP, hs0, Wk1hi, Wk1lo, bk1r, Wk2hi, Wk2lo, bk2r,
                        Wk3hi, Wk3lo, bk3r, S, R)
    s1, c1 = _scatter_cnt_k(msg1.reshape(E, DN), dst, zeros, ones)
    s1p = s1.reshape(2, N // 8, 128)
    c1p = c1.reshape(2, N // 8, 128)
    h1 = _update(s1p, c1p, h0, rootB, cbB)
    hs1 = _gather_k(h1.reshape(N, DN), src).reshape(E // 8, 128)
    msg2 = _msg(a2, hs1, Wk3hi, Wk3lo, bk3r, S, R)
    (s2,) = _scatter_k(msg2.reshape(E, DN), dst, zeros, ones)
    h2 = _update(s2.reshape(2, N // 8, 128), c1p, h1, rootB, cbB)

    hp = jnp.pad(h2.reshape(N, DN), ((0, NP - N), (0, 0)))
    bp = jnp.pad(batch, (0, NP - N), constant_values=-1).reshape(1, NP)
    return _pool(hp, bp, W2, b2r)


# speedup vs baseline: 1.1615x; 1.1615x over previous
import functools

import jax
import jax.numpy as jnp
from jax import lax
from jax.experimental import pallas as pl
from jax.experimental.pallas import tpu as pltpu
from jax.experimental.pallas import tpu_sc as plsc

N = 10000
E = 320000
G = 16
DIM_IN = 128
DN = 16

NW = 32
EPW = E // NW
CH = 2000
NCH = EPW // CH

NB_N = 10
BN = N // NB_N
BE = 3200
NBE = E // BE

NP = 10112


def _f32(x):
    return jnp.dot(x[0], x[1], preferred_element_type=jnp.float32)


def _h0_body(x_ref, w_ref, b_ref, o_ref):
    o_ref[...] = jnp.dot(x_ref[...], w_ref[...],
                         preferred_element_type=jnp.float32) + b_ref[...]


def _h0(xp, W1B, b1B):
    full = lambda a: pl.BlockSpec(a.shape, lambda: tuple(0 for _ in a.shape))
    return pl.pallas_call(
        _h0_body,
        in_specs=[full(xp), full(W1B), full(b1B)],
        out_specs=pl.BlockSpec((N // 8, 128), lambda: (0, 0)),
        out_shape=jax.ShapeDtypeStruct((N // 8, 128), jnp.float32),
    )(xp, W1B, b1B)


def _dot2(a, bhi, blo):
    f = jnp.float32
    abf = a.astype(jnp.bfloat16)
    return (jnp.dot(abf, bhi, preferred_element_type=f)
            + jnp.dot(abf, blo, preferred_element_type=f))


def _l3_w(a2, wk3hi, wk3lo, bk3):
    parts = [
        _dot2(a2[:, e * 96:(e + 1) * 96], wk3hi[...], wk3lo[...])
        for e in range(8)
    ]
    return (jnp.concatenate(parts, axis=1) + bk3[...]).astype(jnp.bfloat16)


def _mlp_msg_body(ea_ref, hs_ref, wk1hi, wk1lo, bk1, wk2hi, wk2lo, bk2,
                  wk3hi, wk3lo, bk3, S, R, a2_out, msg_out):
    bf = jnp.bfloat16
    a1 = jnp.maximum(
        _dot2(ea_ref[...], wk1hi[...], wk1lo[...]) + bk1[...], 0.0)
    a2 = jnp.maximum(
        _dot2(a1, wk2hi[...], wk2lo[...]) + bk2[...], 0.0)
    a2_out[...] = a2
    w = _l3_w(a2, wk3hi, wk3lo, bk3)
    hsbig = jnp.dot(hs_ref[...].astype(bf), S[...],
                    preferred_element_type=jnp.float32)
    msg_out[...] = jnp.dot((hsbig * w).astype(bf), R[...],
                           preferred_element_type=jnp.float32)


def _mlp_msg(ea, hs, Wk1hi, Wk1lo, bk1r, Wk2hi, Wk2lo, bk2r, Wk3hi, Wk3lo,
             bk3r, S, R):
    full = lambda a: pl.BlockSpec(a.shape, lambda i: tuple(0 for _ in a.shape))
    return pl.pallas_call(
        _mlp_msg_body,
        grid=(NBE,),
        in_specs=[
            pl.BlockSpec((BE // 8, 128), lambda i: (i, 0)),
            pl.BlockSpec((BE // 8, 128), lambda i: (i, 0)),
            full(Wk1hi), full(Wk1lo), full(bk1r), full(Wk2hi), full(Wk2lo),
            full(bk2r), full(Wk3hi), full(Wk3lo), full(bk3r), full(S),
            full(R),
        ],
        out_specs=[
            pl.BlockSpec((BE // 8, 8 * 96), lambda i: (i, 0)),
            pl.BlockSpec((BE // 8, 128), lambda i: (i, 0)),
        ],
        out_shape=[
            jax.ShapeDtypeStruct((E // 8, 8 * 96), jnp.float32),
            jax.ShapeDtypeStruct((E // 8, 128), jnp.float32),
        ],
    )(ea, hs, Wk1hi, Wk1lo, bk1r, Wk2hi, Wk2lo, bk2r, Wk3hi, Wk3lo, bk3r,
      S, R)


def _msg_body(a2_ref, hs_ref, wk3hi, wk3lo, bk3, S, R, msg_out):
    bf = jnp.bfloat16
    w = _l3_w(a2_ref[...], wk3hi, wk3lo, bk3)
    hsbig = jnp.dot(hs_ref[...].astype(bf), S[...],
                    preferred_element_type=jnp.float32)
    msg_out[...] = jnp.dot((hsbig * w).astype(bf), R[...],
                           preferred_element_type=jnp.float32)


def _msg(a2, hs, Wk3hi, Wk3lo, bk3r, S, R):
    full = lambda a: pl.BlockSpec(a.shape, lambda i: tuple(0 for _ in a.shape))
    return pl.pallas_call(
        _msg_body,
        grid=(NBE,),
        in_specs=[
            pl.BlockSpec((BE // 8, 8 * 96), lambda i: (i, 0)),
            pl.BlockSpec((BE // 8, 128), lambda i: (i, 0)),
            full(Wk3hi), full(Wk3lo), full(bk3r), full(S), full(R),
        ],
        out_specs=pl.BlockSpec((BE // 8, 128), lambda i: (i, 0)),
        out_shape=jax.ShapeDtypeStruct((E // 8, 128), jnp.float32),
    )(a2, hs, Wk3hi, Wk3lo, bk3r, S, R)


def _update_body(s_ref, c_ref, h_ref, root, cb, o_ref):
    cnt = jnp.maximum(c_ref[0] + c_ref[1], 1.0)
    aggr = (s_ref[0] + s_ref[1]) / cnt
    hr = jnp.dot(h_ref[...], root[...], preferred_element_type=jnp.float32)
    o_ref[...] = jnp.maximum(aggr + hr + cb[...], 0.0)


def _update(sp, cp, hp, rootB, cbB):
    full = lambda a: pl.BlockSpec(a.shape, lambda: tuple(0 for _ in a.shape))
    return pl.pallas_call(
        _update_body,
        in_specs=[full(sp), full(cp), full(hp), full(rootB), full(cbB)],
        out_specs=pl.BlockSpec((N // 8, 128), lambda: (0, 0)),
        out_shape=jax.ShapeDtypeStruct((N // 8, 128), jnp.float32),
    )(sp, cp, hp, rootB, cbB)


def _pool_body(h_ref, b_ref, w2, b2, o_ref):
    ids = lax.broadcasted_iota(jnp.int32, (G, NP), 0)
    oh = (ids == b_ref[...]).astype(jnp.float32)
    s = jnp.dot(oh, h_ref[...], preferred_element_type=jnp.float32)
    cnt = jnp.maximum(jnp.sum(oh, axis=1, keepdims=True), 1.0)
    o_ref[...] = jnp.dot(s / cnt, w2[...],
                         preferred_element_type=jnp.float32) + b2[...]


def _pool(hp, bp, W2, b2r):
    full = lambda a: pl.BlockSpec(a.shape, lambda: tuple(0 for _ in a.shape))
    return pl.pallas_call(
        _pool_body,
        in_specs=[full(hp), full(bp), full(W2), full(b2r)],
        out_specs=pl.BlockSpec((G, 1), lambda: (0, 0)),
        out_shape=jax.ShapeDtypeStruct((G, 1), jnp.float32),
    )(hp, bp, W2, b2r)


_MESH = plsc.VectorSubcoreMesh(core_axis_name="c", subcore_axis_name="s")
_SC_PARAMS = pltpu.CompilerParams(use_tc_tiling_on_sc=False)


@functools.partial(
    pl.kernel,
    out_type=jax.ShapeDtypeStruct((E, DN), jnp.float32),
    mesh=_MESH,
    compiler_params=_SC_PARAMS,
    scratch_types=[
        pltpu.VMEM((CH,), jnp.int32),
        pltpu.VMEM((CH,), jnp.int32),
        pltpu.VMEM((CH, DN), jnp.float32),
        pltpu.VMEM((CH, DN), jnp.float32),
        pltpu.SemaphoreType.DMA,
        pltpu.SemaphoreType.DMA,
    ],
)
def _gather_k(h_hbm, src_hbm, out_hbm, i0, i1, r0, r1, s0, s1):
    cid = lax.axis_index("c")
    sid = lax.axis_index("s")
    wid = sid * 2 + cid
    base = wid * EPW
    idx = [i0, i1]
    rows = [r0, r1]
    sems = [s0, s1]
    descs = [None, None]
    pltpu.sync_copy(src_hbm.at[pl.ds(base, CH)], i0)
    descs[0] = pltpu.async_copy(h_hbm.at[i0], r0, s0)
    for j in range(1, NCH):
        b = j & 1
        pltpu.sync_copy(src_hbm.at[pl.ds(base + j * CH, CH)], idx[b])
        descs[b] = pltpu.async_copy(h_hbm.at[idx[b]], rows[b], sems[b])
        descs[1 - b].wait()
        pltpu.sync_copy(rows[1 - b], out_hbm.at[pl.ds(base + (j - 1) * CH, CH)])
    last = (NCH - 1) & 1
    descs[last].wait()
    pltpu.sync_copy(rows[last], out_hbm.at[pl.ds(base + (NCH - 1) * CH, CH)])


def _make_scatter(with_cnt):
    outs = (jax.ShapeDtypeStruct((2, N, DN), jnp.float32),)
    scratch = [
        pltpu.VMEM((CH,), jnp.int32),
        pltpu.VMEM((CH,), jnp.int32),
        pltpu.VMEM((CH, DN), jnp.float32),
        pltpu.VMEM((CH, DN), jnp.float32),
        pltpu.VMEM_SHARED((N, DN), jnp.float32),
        pltpu.SemaphoreType.DMA,
        pltpu.SemaphoreType.DMA,
    ]
    if with_cnt:
        outs = outs + (jax.ShapeDtypeStruct((2, N, DN), jnp.float32),)
        scratch += [
            pltpu.VMEM((CH, DN), jnp.float32),
            pltpu.VMEM_SHARED((N, DN), jnp.float32),
            pltpu.SemaphoreType.DMA,
            pltpu.SemaphoreType.DMA,
        ]

    @functools.partial(pl.kernel, out_type=outs, mesh=_MESH,
                       compiler_params=_SC_PARAMS, scratch_types=scratch)
    def _scatter_k(msg_hbm, dst_hbm, zeros_hbm, ones_hbm, *rest):
        if with_cnt:
            (s_out, c_out, i0, i1, m0, m1, s_sh, sm0, sm1,
             ones_v, c_sh, sc0, sc1) = rest
            csems = [sc0, sc1]
        else:
            s_out, i0, i1, m0, m1, s_sh, sm0, sm1 = rest
        cid = lax.axis_index("c")
        sid = lax.axis_index("s")
        wid = sid * 2 + cid
        base = wid * EPW

        @pl.when(sid == 0)
        def _():
            pltpu.sync_copy(zeros_hbm, s_sh)
            if with_cnt:
                pltpu.sync_copy(zeros_hbm, c_sh)

        if with_cnt:
            pltpu.sync_copy(ones_hbm, ones_v)
        plsc.subcore_barrier()
        idx = [i0, i1]
        msgv = [m0, m1]
        sems = [sm0, sm1]
        descs = [None, None]
        cdescs = [None, None]
        pltpu.sync_copy(dst_hbm.at[pl.ds(base, CH)], i0)
        pltpu.sync_copy(msg_hbm.at[pl.ds(base, CH)], m0)
        for j in range(NCH):
            b = j & 1
            descs[b] = pltpu.async_copy(msgv[b], s_sh.at[idx[b]], sems[b],
                                        add=True)
            if with_cnt:
                cdescs[b] = pltpu.async_copy(ones_v, c_sh.at[idx[b]],
                                             csems[b], add=True)
            if j + 1 < NCH:
                if descs[1 - b] is not None:
                    descs[1 - b].wait()
                    if with_cnt:
                        cdescs[1 - b].wait()
                pltpu.sync_copy(dst_hbm.at[pl.ds(base + (j + 1) * CH, CH)],
                                idx[1 - b])
                pltpu.sync_copy(msg_hbm.at[pl.ds(base + (j + 1) * CH, CH)],
                                msgv[1 - b])
        for b in range(2):
            if descs[b] is not None:
                descs[b].wait()
                if with_cnt:
                    cdescs[b].wait()
        plsc.subcore_barrier()

        @pl.when(sid == 0)
        def _():
            pltpu.sync_copy(s_sh, s_out.at[cid])
            if with_cnt:
                pltpu.sync_copy(c_sh, c_out.at[cid])

    return _scatter_k


_scatter_cnt_k = _make_scatter(True)
_scatter_k = _make_scatter(False)


def kernel(x, edge_index, edge_attr, batch, W1, b1, Wk1, bk1, Wk2, bk2,
           Wk3, bk3, root, cbias, W2, b2):
    src = edge_index[0]
    dst = edge_index[1]

    b2r = b2.reshape(1, 1)

    j = jnp.arange(DN * DN)
    S0 = (j[None, :] // DN == jnp.arange(DN)[:, None]).astype(jnp.float32)
    R0 = (j[:, None] % DN == jnp.arange(DN)[None, :]).astype(jnp.float32)
    I8 = jnp.eye(8, dtype=jnp.float32)
    kr = lambda W: jnp.kron(I8, W).astype(jnp.bfloat16)
    S = kr(S0)
    R = kr(R0)
    def split(a):
        hi = a.astype(jnp.bfloat16)
        lo = (a - hi.astype(jnp.float32)).astype(jnp.bfloat16)
        return hi, lo

    Wk1hi, Wk1lo = split(jnp.kron(I8, Wk1))
    Wk2hi, Wk2lo = split(jnp.kron(I8, Wk2))
    Wk3hi, Wk3lo = split(Wk3)
    bk1r = jnp.tile(bk1, 8).reshape(1, 8 * 64)
    bk2r = jnp.tile(bk2, 8).reshape(1, 8 * 96)
    bk3r = jnp.tile(bk3, 8).reshape(1, 8 * DN * DN)
    W1B = jnp.kron(I8, W1)
    b1B = jnp.tile(b1, 8).reshape(1, 128)
    rootB = jnp.kron(I8, root)
    cbB = jnp.tile(cbias, 8).reshape(1, 128)

    zeros = jnp.zeros((N // 8, 128), jnp.float32).reshape(N, DN)
    ones = jnp.ones((CH // 8, 128), jnp.float32).reshape(CH, DN)

    eaP = edge_attr.reshape(E // 8, 128)
    xp = x.reshape(N // 8, 8 * DIM_IN)
    h0 = _h0(xp, W1B, b1B)
    hs0 = _gather_k(h0.reshape(N, DN), src).reshape(E // 8, 128)
    a2, msg1 = _mlp_msg(eaP, hs0, Wk1hi, Wk1lo, bk1r, Wk2hi, Wk2lo, bk2r,
                        Wk3hi, Wk3lo, bk3r, S, R)
    s1, c1 = _scatter_cnt_k(msg1.reshape(E, DN), dst, zeros, ones)
    s1p = s1.reshape(2, N // 8, 128)
    c1p = c1.reshape(2, N // 8, 128)
    h1 = _update(s1p, c1p, h0, rootB, cbB)
    hs1 = _gather_k(h1.reshape(N, DN), src).reshape(E // 8, 128)
    msg2 = _msg(a2, hs1, Wk3hi, Wk3lo, bk3r, S, R)
    (s2,) = _scatter_k(msg2.reshape(E, DN), dst, zeros, ones)
    h2 = _update(s2.reshape(2, N // 8, 128), c1p, h1, rootB, cbB)

    hp = jnp.pad(h2.reshape(N, DN), ((0, NP - N), (0, 0)))
    bp = jnp.pad(batch, (0, NP - N), constant_values=-1).reshape(1, NP)
    return _pool(hp, bp, W2, b2r)


# speedup vs baseline: 1.2185x; 1.0490x over previous
import functools

import jax
import jax.numpy as jnp
from jax import lax
from jax.experimental import pallas as pl
from jax.experimental.pallas import tpu as pltpu
from jax.experimental.pallas import tpu_sc as plsc

N = 10000
E = 320000
G = 16
DIM_IN = 128
DN = 16

NW = 32
EPW = E // NW
CH = 2000
NCH = EPW // CH

NB_N = 10
BN = N // NB_N
BE = 6400
NBE = E // BE

NP = 10112


def _f32(x):
    return jnp.dot(x[0], x[1], preferred_element_type=jnp.float32)


def _h0_body(x_ref, whi, wlo, b_ref, o_ref):
    o_ref[...] = _dot2(x_ref[...], whi[...], wlo[...]) + b_ref[...]


def _h0(xp, W1Bhi, W1Blo, b1B):
    full = lambda a: pl.BlockSpec(a.shape, lambda: tuple(0 for _ in a.shape))
    return pl.pallas_call(
        _h0_body,
        in_specs=[full(xp), full(W1Bhi), full(W1Blo), full(b1B)],
        out_specs=pl.BlockSpec((N // 8, 128), lambda: (0, 0)),
        out_shape=jax.ShapeDtypeStruct((N // 8, 128), jnp.float32),
    )(xp, W1Bhi, W1Blo, b1B)


def _dot2(a, bhi, blo):
    f = jnp.float32
    abf = a.astype(jnp.bfloat16)
    return (jnp.dot(abf, bhi, preferred_element_type=f)
            + jnp.dot(abf, blo, preferred_element_type=f))


def _l3_w(a2, wk3hi, wk3lo, bk3):
    parts = [
        _dot2(a2[:, e * 96:(e + 1) * 96], wk3hi[...], wk3lo[...])
        for e in range(8)
    ]
    return (jnp.concatenate(parts, axis=1) + bk3[...]).astype(jnp.bfloat16)


def _mlp_msg_body(ea_ref, hs_ref, wk1hi, wk1lo, bk1, wk2hi, wk2lo, bk2,
                  wk3hi, wk3lo, bk3, S, R, a2_out, msg_out):
    bf = jnp.bfloat16
    a1 = jnp.maximum(
        _dot2(ea_ref[...], wk1hi[...], wk1lo[...]) + bk1[...], 0.0)
    a2 = jnp.maximum(
        _dot2(a1, wk2hi[...], wk2lo[...]) + bk2[...], 0.0)
    a2_out[...] = a2
    w = _l3_w(a2, wk3hi, wk3lo, bk3)
    hsbig = jnp.dot(hs_ref[...].astype(bf), S[...],
                    preferred_element_type=jnp.float32)
    msg_out[...] = jnp.dot((hsbig * w).astype(bf), R[...],
                           preferred_element_type=jnp.float32)


def _mlp_msg(ea, hs, Wk1hi, Wk1lo, bk1r, Wk2hi, Wk2lo, bk2r, Wk3hi, Wk3lo,
             bk3r, S, R):
    full = lambda a: pl.BlockSpec(a.shape, lambda i: tuple(0 for _ in a.shape))
    return pl.pallas_call(
        _mlp_msg_body,
        grid=(NBE,),
        in_specs=[
            pl.BlockSpec((BE // 8, 128), lambda i: (i, 0)),
            pl.BlockSpec((BE // 8, 128), lambda i: (i, 0)),
            full(Wk1hi), full(Wk1lo), full(bk1r), full(Wk2hi), full(Wk2lo),
            full(bk2r), full(Wk3hi), full(Wk3lo), full(bk3r), full(S),
            full(R),
        ],
        out_specs=[
            pl.BlockSpec((BE // 8, 8 * 96), lambda i: (i, 0)),
            pl.BlockSpec((BE // 8, 128), lambda i: (i, 0)),
        ],
        out_shape=[
            jax.ShapeDtypeStruct((E // 8, 8 * 96), jnp.float32),
            jax.ShapeDtypeStruct((E // 8, 128), jnp.float32),
        ],
    )(ea, hs, Wk1hi, Wk1lo, bk1r, Wk2hi, Wk2lo, bk2r, Wk3hi, Wk3lo, bk3r,
      S, R)


def _msg_body(a2_ref, hs_ref, wk3hi, wk3lo, bk3, S, R, msg_out):
    bf = jnp.bfloat16
    w = _l3_w(a2_ref[...], wk3hi, wk3lo, bk3)
    hsbig = jnp.dot(hs_ref[...].astype(bf), S[...],
                    preferred_element_type=jnp.float32)
    msg_out[...] = jnp.dot((hsbig * w).astype(bf), R[...],
                           preferred_element_type=jnp.float32)


def _msg(a2, hs, Wk3hi, Wk3lo, bk3r, S, R):
    full = lambda a: pl.BlockSpec(a.shape, lambda i: tuple(0 for _ in a.shape))
    return pl.pallas_call(
        _msg_body,
        grid=(NBE,),
        in_specs=[
            pl.BlockSpec((BE // 8, 8 * 96), lambda i: (i, 0)),
            pl.BlockSpec((BE // 8, 128), lambda i: (i, 0)),
            full(Wk3hi), full(Wk3lo), full(bk3r), full(S), full(R),
        ],
        out_specs=pl.BlockSpec((BE // 8, 128), lambda i: (i, 0)),
        out_shape=jax.ShapeDtypeStruct((E // 8, 128), jnp.float32),
    )(a2, hs, Wk3hi, Wk3lo, bk3r, S, R)


def _update_body(s_ref, c_ref, h_ref, roothi, rootlo, cb, o_ref):
    cnt = jnp.maximum(c_ref[0] + c_ref[1], 1.0)
    aggr = (s_ref[0] + s_ref[1]) / cnt
    hr = _dot2(h_ref[...], roothi[...], rootlo[...])
    o_ref[...] = jnp.maximum(aggr + hr + cb[...], 0.0)


def _update(sp, cp, hp, rootBhi, rootBlo, cbB):
    full = lambda a: pl.BlockSpec(a.shape, lambda: tuple(0 for _ in a.shape))
    return pl.pallas_call(
        _update_body,
        in_specs=[full(sp), full(cp), full(hp), full(rootBhi), full(rootBlo),
                  full(cbB)],
        out_specs=pl.BlockSpec((N // 8, 128), lambda: (0, 0)),
        out_shape=jax.ShapeDtypeStruct((N // 8, 128), jnp.float32),
    )(sp, cp, hp, rootBhi, rootBlo, cbB)


def _pool_body(h_ref, b_ref, w2hi, w2lo, b2, o_ref):
    h = h_ref[...]
    hhi = h.astype(jnp.bfloat16)
    hlo = (h - hhi.astype(jnp.float32)).astype(jnp.bfloat16)
    ids = lax.broadcasted_iota(jnp.int32, (G, N // 8), 0)
    pooled = jnp.zeros((G, DN), jnp.float32)
    cnt = jnp.zeros((G, 1), jnp.float32)
    for e in range(8):
        oh = (ids == b_ref[e:e + 1, :]).astype(jnp.bfloat16)
        hh = hhi[:, e * DN:(e + 1) * DN]
        hl = hlo[:, e * DN:(e + 1) * DN]
        pooled = (pooled
                  + jnp.dot(oh, hh, preferred_element_type=jnp.float32)
                  + jnp.dot(oh, hl, preferred_element_type=jnp.float32))
        cnt = cnt + jnp.sum(oh.astype(jnp.float32), axis=1, keepdims=True)
    o_ref[...] = _dot2(pooled / jnp.maximum(cnt, 1.0), w2hi[...],
                       w2lo[...]) + b2[...]


def _pool(hp, bt, W2hi, W2lo, b2r):
    full = lambda a: pl.BlockSpec(a.shape, lambda: tuple(0 for _ in a.shape))
    return pl.pallas_call(
        _pool_body,
        in_specs=[full(hp), full(bt), full(W2hi), full(W2lo), full(b2r)],
        out_specs=pl.BlockSpec((G, 1), lambda: (0, 0)),
        out_shape=jax.ShapeDtypeStruct((G, 1), jnp.float32),
    )(hp, bt, W2hi, W2lo, b2r)


_MESH = plsc.VectorSubcoreMesh(core_axis_name="c", subcore_axis_name="s")
_SC_PARAMS = pltpu.CompilerParams(use_tc_tiling_on_sc=False)


@functools.partial(
    pl.kernel,
    out_type=jax.ShapeDtypeStruct((E, DN), jnp.float32),
    mesh=_MESH,
    compiler_params=_SC_PARAMS,
    scratch_types=[
        pltpu.VMEM((CH,), jnp.int32),
        pltpu.VMEM((CH,), jnp.int32),
        pltpu.VMEM((CH, DN), jnp.float32),
        pltpu.VMEM((CH, DN), jnp.float32),
        pltpu.SemaphoreType.DMA,
        pltpu.SemaphoreType.DMA,
    ],
)
def _gather_k(h_hbm, src_hbm, out_hbm, i0, i1, r0, r1, s0, s1):
    cid = lax.axis_index("c")
    sid = lax.axis_index("s")
    wid = sid * 2 + cid
    base = wid * EPW
    idx = [i0, i1]
    rows = [r0, r1]
    sems = [s0, s1]
    descs = [None, None]
    pltpu.sync_copy(src_hbm.at[pl.ds(base, CH)], i0)
    descs[0] = pltpu.async_copy(h_hbm.at[i0], r0, s0)
    for j in range(1, NCH):
        b = j & 1
        pltpu.sync_copy(src_hbm.at[pl.ds(base + j * CH, CH)], idx[b])
        descs[b] = pltpu.async_copy(h_hbm.at[idx[b]], rows[b], sems[b])
        descs[1 - b].wait()
        pltpu.sync_copy(rows[1 - b], out_hbm.at[pl.ds(base + (j - 1) * CH, CH)])
    last = (NCH - 1) & 1
    descs[last].wait()
    pltpu.sync_copy(rows[last], out_hbm.at[pl.ds(base + (NCH - 1) * CH, CH)])


def _make_scatter(with_cnt):
    outs = (jax.ShapeDtypeStruct((2, N, DN), jnp.float32),)
    scratch = [
        pltpu.VMEM((CH,), jnp.int32),
        pltpu.VMEM((CH,), jnp.int32),
        pltpu.VMEM((CH, DN), jnp.float32),
        pltpu.VMEM((CH, DN), jnp.float32),
        pltpu.VMEM_SHARED((N, DN), jnp.float32),
        pltpu.SemaphoreType.DMA,
        pltpu.SemaphoreType.DMA,
    ]
    if with_cnt:
        outs = outs + (jax.ShapeDtypeStruct((2, N, DN), jnp.float32),)
        scratch += [
            pltpu.VMEM((CH, DN), jnp.float32),
            pltpu.VMEM_SHARED((N, DN), jnp.float32),
            pltpu.SemaphoreType.DMA,
            pltpu.SemaphoreType.DMA,
        ]

    @functools.partial(pl.kernel, out_type=outs, mesh=_MESH,
                       compiler_params=_SC_PARAMS, scratch_types=scratch)
    def _scatter_k(msg_hbm, dst_hbm, zeros_hbm, ones_hbm, *rest):
        if with_cnt:
            (s_out, c_out, i0, i1, m0, m1, s_sh, sm0, sm1,
             ones_v, c_sh, sc0, sc1) = rest
            csems = [sc0, sc1]
        else:
            s_out, i0, i1, m0, m1, s_sh, sm0, sm1 = rest
        cid = lax.axis_index("c")
        sid = lax.axis_index("s")
        wid = sid * 2 + cid
        base = wid * EPW

        @pl.when(sid == 0)
        def _():
            pltpu.sync_copy(zeros_hbm, s_sh)
            if with_cnt:
                pltpu.sync_copy(zeros_hbm, c_sh)

        if with_cnt:
            pltpu.sync_copy(ones_hbm, ones_v)
        plsc.subcore_barrier()
        idx = [i0, i1]
        msgv = [m0, m1]
        sems = [sm0, sm1]
        descs = [None, None]
        cdescs = [None, None]
        pltpu.sync_copy(dst_hbm.at[pl.ds(base, CH)], i0)
        pltpu.sync_copy(msg_hbm.at[pl.ds(base, CH)], m0)
        for j in range(NCH):
            b = j & 1
            descs[b] = pltpu.async_copy(msgv[b], s_sh.at[idx[b]], sems[b],
                                        add=True)
            if with_cnt:
                cdescs[b] = pltpu.async_copy(ones_v, c_sh.at[idx[b]],
                                             csems[b], add=True)
            if j + 1 < NCH:
                if descs[1 - b] is not None:
                    descs[1 - b].wait()
                    if with_cnt:
                        cdescs[1 - b].wait()
                pltpu.sync_copy(dst_hbm.at[pl.ds(base + (j + 1) * CH, CH)],
                                idx[1 - b])
                pltpu.sync_copy(msg_hbm.at[pl.ds(base + (j + 1) * CH, CH)],
                                msgv[1 - b])
        for b in range(2):
            if descs[b] is not None:
                descs[b].wait()
                if with_cnt:
                    cdescs[b].wait()
        plsc.subcore_barrier()

        @pl.when(sid == 0)
        def _():
            pltpu.sync_copy(s_sh, s_out.at[cid])
            if with_cnt:
                pltpu.sync_copy(c_sh, c_out.at[cid])

    return _scatter_k


_scatter_cnt_k = _make_scatter(True)
_scatter_k = _make_scatter(False)


def kernel(x, edge_index, edge_attr, batch, W1, b1, Wk1, bk1, Wk2, bk2,
           Wk3, bk3, root, cbias, W2, b2):
    src = edge_index[0]
    dst = edge_index[1]

    b2r = b2.reshape(1, 1)

    j = jnp.arange(DN * DN)
    S0 = (j[None, :] // DN == jnp.arange(DN)[:, None]).astype(jnp.float32)
    R0 = (j[:, None] % DN == jnp.arange(DN)[None, :]).astype(jnp.float32)
    I8 = jnp.eye(8, dtype=jnp.float32)
    kr = lambda W: jnp.kron(I8, W).astype(jnp.bfloat16)
    S = kr(S0)
    R = kr(R0)
    def split(a):
        hi = a.astype(jnp.bfloat16)
        lo = (a - hi.astype(jnp.float32)).astype(jnp.bfloat16)
        return hi, lo

    Wk1hi, Wk1lo = split(jnp.kron(I8, Wk1))
    Wk2hi, Wk2lo = split(jnp.kron(I8, Wk2))
    Wk3hi, Wk3lo = split(Wk3)
    bk1r = jnp.tile(bk1, 8).reshape(1, 8 * 64)
    bk2r = jnp.tile(bk2, 8).reshape(1, 8 * 96)
    bk3r = jnp.tile(bk3, 8).reshape(1, 8 * DN * DN)
    W1Bhi, W1Blo = split(jnp.kron(I8, W1))
    b1B = jnp.tile(b1, 8).reshape(1, 128)
    rootBhi, rootBlo = split(jnp.kron(I8, root))
    cbB = jnp.tile(cbias, 8).reshape(1, 128)
    W2hi, W2lo = split(W2)

    zeros = jnp.zeros((N // 8, 128), jnp.float32).reshape(N, DN)
    ones = jnp.ones((CH // 8, 128), jnp.float32).reshape(CH, DN)

    eaP = edge_attr.reshape(E // 8, 128)
    xp = x.reshape(N // 8, 8 * DIM_IN)
    h0 = _h0(xp, W1Bhi, W1Blo, b1B)
    hs0 = _gather_k(h0.reshape(N, DN), src).reshape(E // 8, 128)
    a2, msg1 = _mlp_msg(eaP, hs0, Wk1hi, Wk1lo, bk1r, Wk2hi, Wk2lo, bk2r,
                        Wk3hi, Wk3lo, bk3r, S, R)
    s1, c1 = _scatter_cnt_k(msg1.reshape(E, DN), dst, zeros, ones)
    s1p = s1.reshape(2, N // 8, 128)
    c1p = c1.reshape(2, N // 8, 128)
    h1 = _update(s1p, c1p, h0, rootBhi, rootBlo, cbB)
    hs1 = _gather_k(h1.reshape(N, DN), src).reshape(E // 8, 128)
    msg2 = _msg(a2, hs1, Wk3hi, Wk3lo, bk3r, S, R)
    (s2,) = _scatter_k(msg2.reshape(E, DN), dst, zeros, ones)
    h2 = _update(s2.reshape(2, N // 8, 128), c1p, h1, rootBhi, rootBlo, cbB)

    bt = batch.reshape(N // 8, 8).T
    return _pool(h2, bt, W2hi, W2lo, b2r)


# speedup vs baseline: 1.2216x; 1.0025x over previous
import functools

import jax
import jax.numpy as jnp
from jax import lax
from jax.experimental import pallas as pl
from jax.experimental.pallas import tpu as pltpu
from jax.experimental.pallas import tpu_sc as plsc

N = 10000
E = 320000
G = 16
DIM_IN = 128
DN = 16

NW = 32
EPW = E // NW
CH = 2000
NCH = EPW // CH

NB_N = 10
BN = N // NB_N
BE = 6400
NBE = E // BE

NP = 10112


def _f32(x):
    return jnp.dot(x[0], x[1], preferred_element_type=jnp.float32)


def _h0_body(x_ref, whi, wlo, b_ref, o_ref):
    o_ref[...] = _dot2(x_ref[...], whi[...], wlo[...]) + b_ref[...]


def _h0(xp, W1Bhi, W1Blo, b1B):
    full = lambda a: pl.BlockSpec(a.shape, lambda: tuple(0 for _ in a.shape))
    return pl.pallas_call(
        _h0_body,
        in_specs=[full(xp), full(W1Bhi), full(W1Blo), full(b1B)],
        out_specs=pl.BlockSpec((N // 8, 128), lambda: (0, 0)),
        out_shape=jax.ShapeDtypeStruct((N // 8, 128), jnp.float32),
    )(xp, W1Bhi, W1Blo, b1B)


def _dot2(a, bhi, blo):
    f = jnp.float32
    abf = a.astype(jnp.bfloat16)
    return (jnp.dot(abf, bhi, preferred_element_type=f)
            + jnp.dot(abf, blo, preferred_element_type=f))


def _l3_w(a2, wk3hi, wk3lo, bk3):
    parts = [
        _dot2(a2[:, e * 96:(e + 1) * 96], wk3hi[...], wk3lo[...])
        for e in range(8)
    ]
    return (jnp.concatenate(parts, axis=1) + bk3[...]).astype(jnp.bfloat16)


def _mlp_msg_body(ea_ref, hs_ref, wk1hi, wk1lo, bk1, wk2hi, wk2lo, bk2,
                  wk3hi, wk3lo, bk3, S, R, a2_out, msg_out):
    bf = jnp.bfloat16
    a1 = jnp.maximum(
        _dot2(ea_ref[...], wk1hi[...], wk1lo[...]) + bk1[...], 0.0)
    a2 = jnp.maximum(
        _dot2(a1, wk2hi[...], wk2lo[...]) + bk2[...], 0.0)
    a2_out[...] = a2
    w = _l3_w(a2, wk3hi, wk3lo, bk3)
    hsbig = jnp.dot(hs_ref[...].astype(bf), S[...],
                    preferred_element_type=jnp.float32)
    msg_out[...] = jnp.dot((hsbig * w).astype(bf), R[...],
                           preferred_element_type=jnp.float32)


def _mlp_msg(ea, hs, Wk1hi, Wk1lo, bk1r, Wk2hi, Wk2lo, bk2r, Wk3hi, Wk3lo,
             bk3r, S, R):
    full = lambda a: pl.BlockSpec(a.shape, lambda i: tuple(0 for _ in a.shape))
    return pl.pallas_call(
        _mlp_msg_body,
        grid=(NBE,),
        in_specs=[
            pl.BlockSpec((BE // 8, 128), lambda i: (i, 0)),
            pl.BlockSpec((BE // 8, 128), lambda i: (i, 0)),
            full(Wk1hi), full(Wk1lo), full(bk1r), full(Wk2hi), full(Wk2lo),
            full(bk2r), full(Wk3hi), full(Wk3lo), full(bk3r), full(S),
            full(R),
        ],
        out_specs=[
            pl.BlockSpec((BE // 8, 8 * 96), lambda i: (i, 0)),
            pl.BlockSpec((BE // 8, 128), lambda i: (i, 0)),
        ],
        out_shape=[
            jax.ShapeDtypeStruct((E // 8, 8 * 96), jnp.float32),
            jax.ShapeDtypeStruct((E // 8, 128), jnp.float32),
        ],
    )(ea, hs, Wk1hi, Wk1lo, bk1r, Wk2hi, Wk2lo, bk2r, Wk3hi, Wk3lo, bk3r,
      S, R)


def _msg_body(a2_ref, hs_ref, wk3hi, wk3lo, bk3, S, R, msg_out):
    bf = jnp.bfloat16
    w = _l3_w(a2_ref[...], wk3hi, wk3lo, bk3)
    hsbig = jnp.dot(hs_ref[...].astype(bf), S[...],
                    preferred_element_type=jnp.float32)
    msg_out[...] = jnp.dot((hsbig * w).astype(bf), R[...],
                           preferred_element_type=jnp.float32)


def _msg(a2, hs, Wk3hi, Wk3lo, bk3r, S, R):
    full = lambda a: pl.BlockSpec(a.shape, lambda i: tuple(0 for _ in a.shape))
    return pl.pallas_call(
        _msg_body,
        grid=(NBE,),
        in_specs=[
            pl.BlockSpec((BE // 8, 8 * 96), lambda i: (i, 0)),
            pl.BlockSpec((BE // 8, 128), lambda i: (i, 0)),
            full(Wk3hi), full(Wk3lo), full(bk3r), full(S), full(R),
        ],
        out_specs=pl.BlockSpec((BE // 8, 128), lambda i: (i, 0)),
        out_shape=jax.ShapeDtypeStruct((E // 8, 128), jnp.float32),
    )(a2, hs, Wk3hi, Wk3lo, bk3r, S, R)


def _update_body(s_ref, c_ref, h_ref, roothi, rootlo, cb, o_ref):
    cnt = jnp.maximum(c_ref[0] + c_ref[1], 1.0)
    aggr = (s_ref[0] + s_ref[1]) / cnt
    hr = _dot2(h_ref[...], roothi[...], rootlo[...])
    o_ref[...] = jnp.maximum(aggr + hr + cb[...], 0.0)


def _update(sp, cp, hp, rootBhi, rootBlo, cbB):
    full = lambda a: pl.BlockSpec(a.shape, lambda: tuple(0 for _ in a.shape))
    return pl.pallas_call(
        _update_body,
        in_specs=[full(sp), full(cp), full(hp), full(rootBhi), full(rootBlo),
                  full(cbB)],
        out_specs=pl.BlockSpec((N // 8, 128), lambda: (0, 0)),
        out_shape=jax.ShapeDtypeStruct((N // 8, 128), jnp.float32),
    )(sp, cp, hp, rootBhi, rootBlo, cbB)


def _pool_body(s_ref, c_ref, h_ref, roothi, rootlo, cb, b_ref, w2hi, w2lo,
               b2, o_ref):
    cnt0 = jnp.maximum(c_ref[0] + c_ref[1], 1.0)
    aggr = (s_ref[0] + s_ref[1]) / cnt0
    h = jnp.maximum(aggr + _dot2(h_ref[...], roothi[...], rootlo[...])
                    + cb[...], 0.0)
    hhi = h.astype(jnp.bfloat16)
    hlo = (h - hhi.astype(jnp.float32)).astype(jnp.bfloat16)
    ids = lax.broadcasted_iota(jnp.int32, (G, N // 8), 0)
    pooled = jnp.zeros((G, DN), jnp.float32)
    cnt = jnp.zeros((G, 1), jnp.float32)
    for e in range(8):
        oh = (ids == b_ref[e:e + 1, :]).astype(jnp.bfloat16)
        hh = hhi[:, e * DN:(e + 1) * DN]
        hl = hlo[:, e * DN:(e + 1) * DN]
        pooled = (pooled
                  + jnp.dot(oh, hh, preferred_element_type=jnp.float32)
                  + jnp.dot(oh, hl, preferred_element_type=jnp.float32))
        cnt = cnt + jnp.sum(oh.astype(jnp.float32), axis=1, keepdims=True)
    o_ref[...] = _dot2(pooled / jnp.maximum(cnt, 1.0), w2hi[...],
                       w2lo[...]) + b2[...]


def _pool(sp, cp, hp, rootBhi, rootBlo, cbB, bt, W2hi, W2lo, b2r):
    full = lambda a: pl.BlockSpec(a.shape, lambda: tuple(0 for _ in a.shape))
    return pl.pallas_call(
        _pool_body,
        in_specs=[full(sp), full(cp), full(hp), full(rootBhi), full(rootBlo),
                  full(cbB), full(bt), full(W2hi), full(W2lo), full(b2r)],
        out_specs=pl.BlockSpec((G, 1), lambda: (0, 0)),
        out_shape=jax.ShapeDtypeStruct((G, 1), jnp.float32),
    )(sp, cp, hp, rootBhi, rootBlo, cbB, bt, W2hi, W2lo, b2r)


_MESH = plsc.VectorSubcoreMesh(core_axis_name="c", subcore_axis_name="s")
_SC_PARAMS = pltpu.CompilerParams(use_tc_tiling_on_sc=False)


@functools.partial(
    pl.kernel,
    out_type=jax.ShapeDtypeStruct((E, DN), jnp.float32),
    mesh=_MESH,
    compiler_params=_SC_PARAMS,
    scratch_types=[
        pltpu.VMEM((CH,), jnp.int32),
        pltpu.VMEM((CH,), jnp.int32),
        pltpu.VMEM((CH, DN), jnp.float32),
        pltpu.VMEM((CH, DN), jnp.float32),
        pltpu.SemaphoreType.DMA,
        pltpu.SemaphoreType.DMA,
    ],
)
def _gather_k(h_hbm, src_hbm, out_hbm, i0, i1, r0, r1, s0, s1):
    cid = lax.axis_index("c")
    sid = lax.axis_index("s")
    wid = sid * 2 + cid
    base = wid * EPW
    idx = [i0, i1]
    rows = [r0, r1]
    sems = [s0, s1]
    descs = [None, None]
    pltpu.sync_copy(src_hbm.at[pl.ds(base, CH)], i0)
    descs[0] = pltpu.async_copy(h_hbm.at[i0], r0, s0)
    for j in range(1, NCH):
        b = j & 1
        pltpu.sync_copy(src_hbm.at[pl.ds(base + j * CH, CH)], idx[b])
        descs[b] = pltpu.async_copy(h_hbm.at[idx[b]], rows[b], sems[b])
        descs[1 - b].wait()
        pltpu.sync_copy(rows[1 - b], out_hbm.at[pl.ds(base + (j - 1) * CH, CH)])
    last = (NCH - 1) & 1
    descs[last].wait()
    pltpu.sync_copy(rows[last], out_hbm.at[pl.ds(base + (NCH - 1) * CH, CH)])


def _make_scatter(with_cnt):
    outs = (jax.ShapeDtypeStruct((2, N, DN), jnp.float32),)
    scratch = [
        pltpu.VMEM((CH,), jnp.int32),
        pltpu.VMEM((CH,), jnp.int32),
        pltpu.VMEM((CH, DN), jnp.float32),
        pltpu.VMEM((CH, DN), jnp.float32),
        pltpu.VMEM_SHARED((N, DN), jnp.float32),
        pltpu.SemaphoreType.DMA,
        pltpu.SemaphoreType.DMA,
    ]
    if with_cnt:
        outs = outs + (jax.ShapeDtypeStruct((2, N, DN), jnp.float32),)
        scratch += [
            pltpu.VMEM((CH, DN), jnp.float32),
            pltpu.VMEM_SHARED((N, DN), jnp.float32),
            pltpu.SemaphoreType.DMA,
            pltpu.SemaphoreType.DMA,
        ]

    @functools.partial(pl.kernel, out_type=outs, mesh=_MESH,
                       compiler_params=_SC_PARAMS, scratch_types=scratch)
    def _scatter_k(msg_hbm, dst_hbm, zeros_hbm, ones_hbm, *rest):
        if with_cnt:
            (s_out, c_out, i0, i1, m0, m1, s_sh, sm0, sm1,
             ones_v, c_sh, sc0, sc1) = rest
            csems = [sc0, sc1]
        else:
            s_out, i0, i1, m0, m1, s_sh, sm0, sm1 = rest
        cid = lax.axis_index("c")
        sid = lax.axis_index("s")
        wid = sid * 2 + cid
        base = wid * EPW

        @pl.when(sid == 0)
        def _():
            pltpu.sync_copy(zeros_hbm, s_sh)
            if with_cnt:
                pltpu.sync_copy(zeros_hbm, c_sh)

        if with_cnt:
            pltpu.sync_copy(ones_hbm, ones_v)
        plsc.subcore_barrier()
        idx = [i0, i1]
        msgv = [m0, m1]
        sems = [sm0, sm1]
        descs = [None, None]
        cdescs = [None, None]
        pltpu.sync_copy(dst_hbm.at[pl.ds(base, CH)], i0)
        pltpu.sync_copy(msg_hbm.at[pl.ds(base, CH)], m0)
        for j in range(NCH):
            b = j & 1
            descs[b] = pltpu.async_copy(msgv[b], s_sh.at[idx[b]], sems[b],
                                        add=True)
            if with_cnt:
                cdescs[b] = pltpu.async_copy(ones_v, c_sh.at[idx[b]],
                                             csems[b], add=True)
            if j + 1 < NCH:
                if descs[1 - b] is not None:
                    descs[1 - b].wait()
                    if with_cnt:
                        cdescs[1 - b].wait()
                pltpu.sync_copy(dst_hbm.at[pl.ds(base + (j + 1) * CH, CH)],
                                idx[1 - b])
                pltpu.sync_copy(msg_hbm.at[pl.ds(base + (j + 1) * CH, CH)],
                                msgv[1 - b])
        for b in range(2):
            if descs[b] is not None:
                descs[b].wait()
                if with_cnt:
                    cdescs[b].wait()
        plsc.subcore_barrier()

        @pl.when(sid == 0)
        def _():
            pltpu.sync_copy(s_sh, s_out.at[cid])
            if with_cnt:
                pltpu.sync_copy(c_sh, c_out.at[cid])

    return _scatter_k


_scatter_cnt_k = _make_scatter(True)
_scatter_k = _make_scatter(False)


def kernel(x, edge_index, edge_attr, batch, W1, b1, Wk1, bk1, Wk2, bk2,
           Wk3, bk3, root, cbias, W2, b2):
    src = edge_index[0]
    dst = edge_index[1]

    b2r = b2.reshape(1, 1)

    j = jnp.arange(DN * DN)
    S0 = (j[None, :] // DN == jnp.arange(DN)[:, None]).astype(jnp.float32)
    R0 = (j[:, None] % DN == jnp.arange(DN)[None, :]).astype(jnp.float32)
    I8 = jnp.eye(8, dtype=jnp.float32)
    kr = lambda W: jnp.kron(I8, W).astype(jnp.bfloat16)
    S = kr(S0)
    R = kr(R0)
    def split(a):
        hi = a.astype(jnp.bfloat16)
        lo = (a - hi.astype(jnp.float32)).astype(jnp.bfloat16)
        return hi, lo

    Wk1hi, Wk1lo = split(jnp.kron(I8, Wk1))
    Wk2hi, Wk2lo = split(jnp.kron(I8, Wk2))
    Wk3hi, Wk3lo = split(Wk3)
    bk1r = jnp.tile(bk1, 8).reshape(1, 8 * 64)
    bk2r = jnp.tile(bk2, 8).reshape(1, 8 * 96)
    bk3r = jnp.tile(bk3, 8).reshape(1, 8 * DN * DN)
    W1Bhi, W1Blo = split(jnp.kron(I8, W1))
    b1B = jnp.tile(b1, 8).reshape(1, 128)
    rootBhi, rootBlo = split(jnp.kron(I8, root))
    cbB = jnp.tile(cbias, 8).reshape(1, 128)
    W2hi, W2lo = split(W2)

    zeros = jnp.zeros((N // 8, 128), jnp.float32).reshape(N, DN)
    ones = jnp.ones((CH // 8, 128), jnp.float32).reshape(CH, DN)

    eaP = edge_attr.reshape(E // 8, 128)
    xp = x.reshape(N // 8, 8 * DIM_IN)
    h0 = _h0(xp, W1Bhi, W1Blo, b1B)
    hs0 = _gather_k(h0.reshape(N, DN), src).reshape(E // 8, 128)
    a2, msg1 = _mlp_msg(eaP, hs0, Wk1hi, Wk1lo, bk1r, Wk2hi, Wk2lo, bk2r,
                        Wk3hi, Wk3lo, bk3r, S, R)
    s1, c1 = _scatter_cnt_k(msg1.reshape(E, DN), dst, zeros, ones)
    s1p = s1.reshape(2, N // 8, 128)
    c1p = c1.reshape(2, N // 8, 128)
    h1 = _update(s1p, c1p, h0, rootBhi, rootBlo, cbB)
    hs1 = _gather_k(h1.reshape(N, DN), src).reshape(E // 8, 128)
    msg2 = _msg(a2, hs1, Wk3hi, Wk3lo, bk3r, S, R)
    (s2,) = _scatter_k(msg2.reshape(E, DN), dst, zeros, ones)
    bt = batch.reshape(N // 8, 8).T
    return _pool(s2.reshape(2, N // 8, 128), c1p, h1, rootBhi, rootBlo, cbB,
                 bt, W2hi, W2lo, b2r)
